# baseline (speedup 1.0000x reference)
import functools

import jax
import jax.numpy as jnp
from jax import lax
from jax.experimental import pallas as pl
from jax.experimental.pallas import tpu as pltpu
from jax.experimental.pallas import tpu_sc as plsc

_NC = 2
_NS = 16
_NW = _NC * _NS


def _pre_body(x_ref, ws_ref, wr_ref, xs_ref, xr_ref):
    x = x_ref[...]
    xs_ref[...] = jnp.dot(x, ws_ref[...], preferred_element_type=jnp.float32)
    xr_ref[...] = jnp.dot(x, wr_ref[...], preferred_element_type=jnp.float32)


def _pre(x, w_s, w_r):
    n, d = x.shape
    out = jax.ShapeDtypeStruct((n, d), jnp.float32)
    return pl.pallas_call(
        _pre_body,
        out_shape=(out, out),
    )(x, w_s, w_r)


def _gather(xs, xr, senders, receivers, e0, e_seg):
    d = xs.shape[1]
    blk = 200
    per_w = e_seg // _NW
    chunks = per_w // blk
    assert chunks * blk == per_w and per_w * _NW == e_seg
    assert chunks % 2 == 1

    mesh = plsc.VectorSubcoreMesh(core_axis_name="c", subcore_axis_name="s")
    out = jax.ShapeDtypeStruct((e_seg, d), jnp.float32)
    idx_t = pltpu.VMEM((blk,), jnp.int32)
    row_t = pltpu.VMEM((blk, d), jnp.float32)

    @functools.partial(
        pl.kernel,
        out_type=(out, out),
        mesh=mesh,
        scratch_types=[idx_t, idx_t, idx_t, idx_t, row_t, row_t, row_t, row_t,
                       pltpu.SemaphoreType.DMA, pltpu.SemaphoreType.DMA],
    )
    def k(xs_hbm, xr_hbm, s_hbm, r_hbm, gs_hbm, gr_hbm,
          sidx0, sidx1, ridx0, ridx1, srow0, srow1, rrow0, rrow1,
          gsem, wsem):
        wid = lax.axis_index("s") * _NC + lax.axis_index("c")
        base = wid * per_w
        sidx = (sidx0, sidx1)
        ridx = (ridx0, ridx1)
        srow = (srow0, srow1)
        rrow = (rrow0, rrow1)

        def issue(chunk, b):
            off = base + chunk * blk
            pltpu.sync_copy(s_hbm.at[pl.ds(e0 + off, blk)], sidx[b])
            pltpu.sync_copy(r_hbm.at[pl.ds(e0 + off, blk)], ridx[b])
            pltpu.async_copy(xs_hbm.at[sidx[b]], srow[b], gsem)
            pltpu.async_copy(xr_hbm.at[ridx[b]], rrow[b], gsem)

        def wait_gather(b):
            pltpu.make_async_copy(xs_hbm.at[sidx[b]], srow[b], gsem).wait()
            pltpu.make_async_copy(xr_hbm.at[ridx[b]], rrow[b], gsem).wait()

        def writeback(chunk, b):
            off = base + chunk * blk
            pltpu.async_copy(srow[b], gs_hbm.at[pl.ds(off, blk)], wsem)
            pltpu.async_copy(rrow[b], gr_hbm.at[pl.ds(off, blk)], wsem)

        def wait_writeback(chunk, b):
            off = base + chunk * blk
            pltpu.make_async_copy(
                srow[b], gs_hbm.at[pl.ds(off, blk)], wsem).wait()
            pltpu.make_async_copy(
                rrow[b], gr_hbm.at[pl.ds(off, blk)], wsem).wait()

        issue(0, 0)
        issue(1, 1)

        @pl.loop(0, chunks - 2, step=2)
        def _(ci):
            wait_gather(0)
            writeback(ci, 0)
            wait_gather(1)
            writeback(ci + 1, 1)
            wait_writeback(ci, 0)
            issue(ci + 2, 0)
            wait_writeback(ci + 1, 1)

            @pl.when(ci + 3 < chunks)
            def _():
                issue(ci + 3, 1)

        wait_gather(0)
        writeback(chunks - 1, 0)
        wait_writeback(chunks - 1, 0)

    return k(xs, xr, senders, receivers)


def _edge_body(gs_ref, gr_ref, ef_ref, w1_ref, b1_ref, w2_ref, b2_ref, out_ref):
    ef = ef_ref[...]
    g = gs_ref[...] + gr_ref[...]
    h = g + b1_ref[...] + jnp.dot(
        ef.astype(jnp.bfloat16), w1_ref[...],
        preferred_element_type=jnp.float32)
    h = jnp.maximum(h, 0.0)
    out_ref[...] = ef + b2_ref[...] + jnp.dot(
        h.astype(jnp.bfloat16), w2_ref[...],
        preferred_element_type=jnp.float32)


def _edge(gs, gr, ef, w1, b1, w2, b2, e0, e_seg):
    d = ef.shape[1]
    blk = 2000
    grid = e_seg // blk
    assert grid * blk == e_seg and e0 % blk == 0
    seg = e0 // blk
    row = pl.BlockSpec((blk, d), lambda i: (i, 0))
    row_seg = pl.BlockSpec((blk, d), lambda i: (i + seg, 0))
    full = pl.BlockSpec((d, d), lambda i: (0, 0))
    vec = pl.BlockSpec((1, d), lambda i: (0, 0))
    return pl.pallas_call(
        _edge_body,
        grid=(grid,),
        in_specs=[row, row, row_seg, full, vec, full, vec],
        out_specs=row,
        out_shape=jax.ShapeDtypeStruct((e_seg, d), jnp.float32),
    )(gs, gr, ef, w1, b1, w2, b2)


def _scatter(ne_a, ne_b, receivers, zeros_nd):
    per_core, d = ne_a.shape
    n = zeros_nd.shape[0]
    blk = 80
    per_tile = per_core // _NS
    chunks = per_tile // blk
    rows_per_tile = n // _NS
    assert chunks * blk == per_tile and rows_per_tile * _NS == n
    assert chunks % 2 == 1
    assert rows_per_tile % 8 == 0

    mesh = plsc.VectorSubcoreMesh(core_axis_name="c", subcore_axis_name="s")
    out = jax.ShapeDtypeStruct((n, d), jnp.float32)
    idx_t = pltpu.VMEM((blk,), jnp.int32)
    row_t = pltpu.VMEM((blk, d), jnp.float32)

    @functools.partial(
        pl.kernel,
        out_type=(out, out),
        mesh=mesh,
        scratch_types=[
            pltpu.VMEM_SHARED((n, d), jnp.float32),
            idx_t, idx_t, row_t, row_t,
            pltpu.SemaphoreType.DMA, pltpu.SemaphoreType.DMA,
        ],
    )
    def k(nea_hbm, neb_hbm, r_hbm, z_hbm, a0_hbm, a1_hbm, shared,
          eidx0, eidx1, erow0, erow1, lsem, asem):
        c = lax.axis_index("c")
        s = lax.axis_index("s")
        r0 = s * rows_per_tile
        pltpu.sync_copy(z_hbm.at[pl.ds(r0, rows_per_tile)],
                        shared.at[pl.ds(r0, rows_per_tile)])
        plsc.subcore_barrier()

        base = s * per_tile
        eidx = (eidx0, eidx1)
        erow = (erow0, erow1)

        def load(chunk, b):
            off = base + chunk * blk
            pltpu.async_copy(r_hbm.at[pl.ds(c * per_core + off, blk)],
                             eidx[b], lsem)

            @pl.when(c == 0)
            def _():
                pltpu.async_copy(nea_hbm.at[pl.ds(off, blk)], erow[b], lsem)

            @pl.when(c == 1)
            def _():
                pltpu.async_copy(neb_hbm.at[pl.ds(off, blk)], erow[b], lsem)

        def wait_load(chunk, b):
            off = base + chunk * blk
            pltpu.make_async_copy(
                r_hbm.at[pl.ds(off, blk)], eidx[b], lsem).wait()
            pltpu.make_async_copy(
                nea_hbm.at[pl.ds(off, blk)], erow[b], lsem).wait()

        def add(b):
            pltpu.async_copy(erow[b], shared.at[eidx[b]], asem, add=True)

        def wait_add(b):
            pltpu.make_async_copy(erow[b], shared.at[eidx[b]], asem).wait()

        load(0, 0)
        load(1, 1)

        @pl.loop(0, chunks - 2, step=2)
        def _(ci):
            wait_load(ci, 0)
            add(0)
            wait_load(ci + 1, 1)
            add(1)
            wait_add(0)
            load(ci + 2, 0)
            wait_add(1)

            @pl.when(ci + 3 < chunks)
            def _():
                load(ci + 3, 1)

        wait_load(chunks - 1, 0)
        add(0)
        wait_add(0)

        plsc.subcore_barrier()

        @pl.when(c == 0)
        def _():
            pltpu.sync_copy(shared.at[pl.ds(r0, rows_per_tile)],
                            a0_hbm.at[pl.ds(r0, rows_per_tile)])

        @pl.when(c == 1)
        def _():
            pltpu.sync_copy(shared.at[pl.ds(r0, rows_per_tile)],
                            a1_hbm.at[pl.ds(r0, rows_per_tile)])

    return k(ne_a, ne_b, receivers, zeros_nd)


def _node_body(x_ref, a0_ref, a1_ref, w1x_ref, w1a_ref, b1_ref, w2_ref,
               b2_ref, out_ref):
    x = x_ref[...]
    agg = a0_ref[...] + a1_ref[...]
    h = b1_ref[...] + jnp.dot(x, w1x_ref[...],
                              preferred_element_type=jnp.float32)
    h = h + jnp.dot(agg, w1a_ref[...], preferred_element_type=jnp.float32)
    h = jnp.maximum(h, 0.0)
    out_ref[...] = x + b2_ref[...] + jnp.dot(
        h, w2_ref[...], preferred_element_type=jnp.float32)


def _node(x, a0, a1, w1x, w1a, b1, w2, b2):
    n, d = x.shape
    blk = 1000
    grid = n // blk
    assert grid * blk == n
    row = pl.BlockSpec((blk, d), lambda i: (i, 0))
    full = pl.BlockSpec((d, d), lambda i: (0, 0))
    vec = pl.BlockSpec((1, d), lambda i: (0, 0))
    return pl.pallas_call(
        _node_body,
        grid=(grid,),
        in_specs=[row, row, row, full, full, vec, full, vec],
        out_specs=row,
        out_shape=jax.ShapeDtypeStruct((n, d), jnp.float32),
    )(x, a0, a1, w1x, w1a, b1, w2, b2)


def kernel(x, edge_features, W1_e, b1_e, W2_e, b2_e,
           W1_n, b1_n, W2_n, b2_n, senders, receivers):
    n, d = x.shape
    e = senders.shape[0]
    senders = senders.astype(jnp.int32)
    receivers = receivers.astype(jnp.int32)
    w1_s, w1_r, w1_ef = W1_e[:d], W1_e[d:2 * d], W1_e[2 * d:]
    w1_ef_b = w1_ef.astype(jnp.bfloat16)
    w2_e_b = W2_e.astype(jnp.bfloat16)
    b1_e2 = b1_e.reshape(1, d)
    b2_e2 = b2_e.reshape(1, d)
    b1_n2 = b1_n.reshape(1, d)
    b2_n2 = b2_n.reshape(1, d)
    w1_nx, w1_na = W1_n[:d], W1_n[d:]
    n_pad = ((n + 8 * _NS - 1) // (8 * _NS)) * (8 * _NS)
    zeros_nd = jnp.zeros((n_pad, d), jnp.float32)
    half = e // 2

    xs, xr = _pre(x, w1_s, w1_r)
    gs_a, gr_a = _gather(xs, xr, senders, receivers, 0, half)
    ne_a = _edge(gs_a, gr_a, edge_features, w1_ef_b, b1_e2, w2_e_b, b2_e2,
                 0, half)
    gs_b, gr_b = _gather(xs, xr, senders, receivers, half, half)
    ne_b = _edge(gs_b, gr_b, edge_features, w1_ef_b, b1_e2, w2_e_b, b2_e2,
                 half, half)
    a0, a1 = _scatter(ne_a, ne_b, receivers, zeros_nd)
    new_e = jnp.concatenate([ne_a, ne_b], axis=0)
    new_x = _node(x, a0, a1, w1_nx, w1_na, b1_n2, W2_n, b2_n2)
    return (new_x, new_e)


# speedup vs baseline: 1.0517x; 1.0517x over previous
import functools

import jax
import jax.numpy as jnp
from jax import lax
from jax.experimental import pallas as pl
from jax.experimental.pallas import tpu as pltpu
from jax.experimental.pallas import tpu_sc as plsc

_NC = 2
_NS = 16
_NW = _NC * _NS


def _pre_body(x_ref, ws_ref, wr_ref, xs_ref, xr_ref):
    x = x_ref[...]
    xs_ref[...] = jnp.dot(x, ws_ref[...], preferred_element_type=jnp.float32)
    xr_ref[...] = jnp.dot(x, wr_ref[...], preferred_element_type=jnp.float32)


def _pre(x, w_s, w_r):
    n, d = x.shape
    out = jax.ShapeDtypeStruct((n, d), jnp.float32)
    return pl.pallas_call(
        _pre_body,
        out_shape=(out, out),
    )(x, w_s, w_r)


def _gather(xs, xr, senders, receivers, e0, e_seg):
    d = xs.shape[1]
    blk = 200
    per_w = e_seg // _NW
    chunks = per_w // blk
    assert chunks * blk == per_w and per_w * _NW == e_seg
    assert chunks % 2 == 1

    mesh = plsc.VectorSubcoreMesh(core_axis_name="c", subcore_axis_name="s")
    out = jax.ShapeDtypeStruct((e_seg, d), jnp.float32)
    idx_t = pltpu.VMEM((blk,), jnp.int32)
    row_t = pltpu.VMEM((blk, d), jnp.float32)

    @functools.partial(
        pl.kernel,
        out_type=(out, out),
        mesh=mesh,
        scratch_types=[idx_t, idx_t, idx_t, idx_t, row_t, row_t, row_t, row_t,
                       pltpu.SemaphoreType.DMA, pltpu.SemaphoreType.DMA],
    )
    def k(xs_hbm, xr_hbm, s_hbm, r_hbm, gs_hbm, gr_hbm,
          sidx0, sidx1, ridx0, ridx1, srow0, srow1, rrow0, rrow1,
          gsem, wsem):
        wid = lax.axis_index("s") * _NC + lax.axis_index("c")
        base = wid * per_w
        sidx = (sidx0, sidx1)
        ridx = (ridx0, ridx1)
        srow = (srow0, srow1)
        rrow = (rrow0, rrow1)

        def issue(chunk, b):
            off = base + chunk * blk
            pltpu.sync_copy(s_hbm.at[pl.ds(e0 + off, blk)], sidx[b])
            pltpu.sync_copy(r_hbm.at[pl.ds(e0 + off, blk)], ridx[b])
            pltpu.async_copy(xs_hbm.at[sidx[b]], srow[b], gsem)
            pltpu.async_copy(xr_hbm.at[ridx[b]], rrow[b], gsem)

        def wait_gather(b):
            pltpu.make_async_copy(xs_hbm.at[sidx[b]], srow[b], gsem).wait()
            pltpu.make_async_copy(xr_hbm.at[ridx[b]], rrow[b], gsem).wait()

        def writeback(chunk, b):
            off = base + chunk * blk
            pltpu.async_copy(srow[b], gs_hbm.at[pl.ds(off, blk)], wsem)
            pltpu.async_copy(rrow[b], gr_hbm.at[pl.ds(off, blk)], wsem)

        def wait_writeback(chunk, b):
            off = base + chunk * blk
            pltpu.make_async_copy(
                srow[b], gs_hbm.at[pl.ds(off, blk)], wsem).wait()
            pltpu.make_async_copy(
                rrow[b], gr_hbm.at[pl.ds(off, blk)], wsem).wait()

        issue(0, 0)
        issue(1, 1)

        @pl.loop(0, chunks - 2, step=2)
        def _(ci):
            wait_gather(0)
            writeback(ci, 0)
            wait_gather(1)
            writeback(ci + 1, 1)
            wait_writeback(ci, 0)
            issue(ci + 2, 0)
            wait_writeback(ci + 1, 1)

            @pl.when(ci + 3 < chunks)
            def _():
                issue(ci + 3, 1)

        wait_gather(0)
        writeback(chunks - 1, 0)
        wait_writeback(chunks - 1, 0)

    return k(xs, xr, senders, receivers)


def _edge_body(gs_ref, gr_ref, ef_ref, w1_ref, b1_ref, w2_ref, b2_ref, out_ref):
    ef = ef_ref[...]
    g = gs_ref[...] + gr_ref[...]
    h = g + b1_ref[...] + jnp.dot(
        ef.astype(jnp.bfloat16), w1_ref[...],
        preferred_element_type=jnp.float32)
    h = jnp.maximum(h, 0.0)
    out_ref[...] = ef + b2_ref[...] + jnp.dot(
        h.astype(jnp.bfloat16), w2_ref[...],
        preferred_element_type=jnp.float32)


def _edge(gs, gr, ef, w1, b1, w2, b2, e0, e_seg):
    d = ef.shape[1]
    blk = 2000
    grid = e_seg // blk
    assert grid * blk == e_seg and e0 % blk == 0
    seg = e0 // blk
    row = pl.BlockSpec((blk, d), lambda i: (i, 0))
    row_seg = pl.BlockSpec((blk, d), lambda i: (i + seg, 0))
    full = pl.BlockSpec((d, d), lambda i: (0, 0))
    vec = pl.BlockSpec((1, d), lambda i: (0, 0))
    return pl.pallas_call(
        _edge_body,
        grid=(grid,),
        in_specs=[row, row, row_seg, full, vec, full, vec],
        out_specs=row,
        out_shape=jax.ShapeDtypeStruct((e_seg, d), jnp.float32),
    )(gs, gr, ef, w1, b1, w2, b2)


def _scatter(ne_a, ne_b, receivers, zeros_nd):
    per_core, d = ne_a.shape
    n = zeros_nd.shape[0]
    blk = 80
    per_tile = per_core // _NS
    chunks = per_tile // blk
    rows_per_tile = n // _NS
    assert chunks * blk == per_tile and rows_per_tile * _NS == n
    assert chunks % 2 == 1
    assert rows_per_tile % 8 == 0

    mesh = plsc.VectorSubcoreMesh(core_axis_name="c", subcore_axis_name="s")
    out = jax.ShapeDtypeStruct((n, d), jnp.float32)
    out_ne = jax.ShapeDtypeStruct((2 * per_core, d), jnp.float32)
    idx_t = pltpu.VMEM((blk,), jnp.int32)
    row_t = pltpu.VMEM((blk, d), jnp.float32)

    @functools.partial(
        pl.kernel,
        out_type=(out, out, out_ne),
        mesh=mesh,
        scratch_types=[
            pltpu.VMEM_SHARED((n, d), jnp.float32),
            idx_t, idx_t, row_t, row_t,
            pltpu.SemaphoreType.DMA, pltpu.SemaphoreType.DMA,
            pltpu.SemaphoreType.DMA,
        ],
    )
    def k(nea_hbm, neb_hbm, r_hbm, z_hbm, a0_hbm, a1_hbm, ne_hbm, shared,
          eidx0, eidx1, erow0, erow1, lsem, asem, wsem):
        c = lax.axis_index("c")
        s = lax.axis_index("s")
        r0 = s * rows_per_tile
        pltpu.sync_copy(z_hbm.at[pl.ds(r0, rows_per_tile)],
                        shared.at[pl.ds(r0, rows_per_tile)])
        plsc.subcore_barrier()

        base = s * per_tile
        eidx = (eidx0, eidx1)
        erow = (erow0, erow1)

        def load(chunk, b):
            off = base + chunk * blk
            pltpu.async_copy(r_hbm.at[pl.ds(c * per_core + off, blk)],
                             eidx[b], lsem)

            @pl.when(c == 0)
            def _():
                pltpu.async_copy(nea_hbm.at[pl.ds(off, blk)], erow[b], lsem)

            @pl.when(c == 1)
            def _():
                pltpu.async_copy(neb_hbm.at[pl.ds(off, blk)], erow[b], lsem)

        def wait_load(chunk, b):
            off = base + chunk * blk
            pltpu.make_async_copy(
                r_hbm.at[pl.ds(off, blk)], eidx[b], lsem).wait()
            pltpu.make_async_copy(
                nea_hbm.at[pl.ds(off, blk)], erow[b], lsem).wait()

        def add(b):
            pltpu.async_copy(erow[b], shared.at[eidx[b]], asem, add=True)

        def wait_add(b):
            pltpu.make_async_copy(erow[b], shared.at[eidx[b]], asem).wait()

        def emit(chunk, b):
            off = c * per_core + base + chunk * blk
            pltpu.async_copy(erow[b], ne_hbm.at[pl.ds(off, blk)], wsem)

        def wait_emit(chunk, b):
            off = c * per_core + base + chunk * blk
            pltpu.make_async_copy(
                erow[b], ne_hbm.at[pl.ds(off, blk)], wsem).wait()

        load(0, 0)
        load(1, 1)

        @pl.loop(0, chunks - 2, step=2)
        def _(ci):
            wait_load(ci, 0)
            add(0)
            emit(ci, 0)
            wait_load(ci + 1, 1)
            add(1)
            emit(ci + 1, 1)
            wait_add(0)
            wait_emit(ci, 0)
            load(ci + 2, 0)
            wait_add(1)

            @pl.when(ci + 3 < chunks)
            def _():
                wait_emit(ci + 1, 1)
                load(ci + 3, 1)

        wait_emit(chunks - 2, 1)
        wait_load(chunks - 1, 0)
        add(0)
        emit(chunks - 1, 0)
        wait_add(0)
        wait_emit(chunks - 1, 0)

        plsc.subcore_barrier()

        @pl.when(c == 0)
        def _():
            pltpu.sync_copy(shared.at[pl.ds(r0, rows_per_tile)],
                            a0_hbm.at[pl.ds(r0, rows_per_tile)])

        @pl.when(c == 1)
        def _():
            pltpu.sync_copy(shared.at[pl.ds(r0, rows_per_tile)],
                            a1_hbm.at[pl.ds(r0, rows_per_tile)])

    return k(ne_a, ne_b, receivers, zeros_nd)


def _node_body(x_ref, a0_ref, a1_ref, w1x_ref, w1a_ref, b1_ref, w2_ref,
               b2_ref, out_ref):
    x = x_ref[...]
    agg = a0_ref[...] + a1_ref[...]
    h = b1_ref[...] + jnp.dot(x, w1x_ref[...],
                              preferred_element_type=jnp.float32)
    h = h + jnp.dot(agg, w1a_ref[...], preferred_element_type=jnp.float32)
    h = jnp.maximum(h, 0.0)
    out_ref[...] = x + b2_ref[...] + jnp.dot(
        h, w2_ref[...], preferred_element_type=jnp.float32)


def _node(x, a0, a1, w1x, w1a, b1, w2, b2):
    n, d = x.shape
    blk = 1000
    grid = n // blk
    assert grid * blk == n
    row = pl.BlockSpec((blk, d), lambda i: (i, 0))
    full = pl.BlockSpec((d, d), lambda i: (0, 0))
    vec = pl.BlockSpec((1, d), lambda i: (0, 0))
    return pl.pallas_call(
        _node_body,
        grid=(grid,),
        in_specs=[row, row, row, full, full, vec, full, vec],
        out_specs=row,
        out_shape=jax.ShapeDtypeStruct((n, d), jnp.float32),
    )(x, a0, a1, w1x, w1a, b1, w2, b2)


def kernel(x, edge_features, W1_e, b1_e, W2_e, b2_e,
           W1_n, b1_n, W2_n, b2_n, senders, receivers):
    n, d = x.shape
    e = senders.shape[0]
    senders = senders.astype(jnp.int32)
    receivers = receivers.astype(jnp.int32)
    w1_s, w1_r, w1_ef = W1_e[:d], W1_e[d:2 * d], W1_e[2 * d:]
    w1_ef_b = w1_ef.astype(jnp.bfloat16)
    w2_e_b = W2_e.astype(jnp.bfloat16)
    b1_e2 = b1_e.reshape(1, d)
    b2_e2 = b2_e.reshape(1, d)
    b1_n2 = b1_n.reshape(1, d)
    b2_n2 = b2_n.reshape(1, d)
    w1_nx, w1_na = W1_n[:d], W1_n[d:]
    n_pad = ((n + 8 * _NS - 1) // (8 * _NS)) * (8 * _NS)
    zeros_nd = jnp.zeros((n_pad, d), jnp.float32)
    half = e // 2

    xs, xr = _pre(x, w1_s, w1_r)
    gs_a, gr_a = _gather(xs, xr, senders, receivers, 0, half)
    ne_a = _edge(gs_a, gr_a, edge_features, w1_ef_b, b1_e2, w2_e_b, b2_e2,
                 0, half)
    gs_b, gr_b = _gather(xs, xr, senders, receivers, half, half)
    ne_b = _edge(gs_b, gr_b, edge_features, w1_ef_b, b1_e2, w2_e_b, b2_e2,
                 half, half)
    a0, a1, new_e = _scatter(ne_a, ne_b, receivers, zeros_nd)
    new_x = _node(x, a0, a1, w1_nx, w1_na, b1_n2, W2_n, b2_n2)
    return (new_x, new_e)


# speedup vs baseline: 1.1888x; 1.1304x over previous
import functools

import jax
import jax.numpy as jnp
from jax import lax
from jax.experimental import pallas as pl
from jax.experimental.pallas import tpu as pltpu
from jax.experimental.pallas import tpu_sc as plsc

_NC = 2
_NS = 16
_NW = _NC * _NS


def _pre_body(x_ref, ws_ref, wr_ref, xs_ref, xr_ref):
    x = x_ref[...]
    xs_ref[...] = jnp.dot(x, ws_ref[...], preferred_element_type=jnp.float32)
    xr_ref[...] = jnp.dot(x, wr_ref[...], preferred_element_type=jnp.float32)


def _pre(x, w_s, w_r):
    n, d = x.shape
    out = jax.ShapeDtypeStruct((n, d), jnp.float32)
    return pl.pallas_call(
        _pre_body,
        out_shape=(out, out),
    )(x, w_s, w_r)


def _gather_sum(xs, xr, senders, receivers, e0, e_seg):
    d = xs.shape[1]
    blk = 200
    per_w = e_seg // _NW
    chunks = per_w // blk
    assert chunks * blk == per_w and per_w * _NW == e_seg
    assert chunks % 2 == 1

    mesh = plsc.VectorSubcoreMesh(core_axis_name="c", subcore_axis_name="s")
    out = jax.ShapeDtypeStruct((e_seg, d), jnp.float32)
    idx_t = pltpu.VMEM((blk,), jnp.int32)
    row_t = pltpu.VMEM((blk, d), jnp.float32)

    @functools.partial(
        pl.kernel,
        out_type=out,
        mesh=mesh,
        scratch_types=[idx_t, idx_t, idx_t, idx_t, row_t, row_t, row_t, row_t,
                       pltpu.SemaphoreType.DMA, pltpu.SemaphoreType.DMA],
    )
    def k(xs_hbm, xr_hbm, s_hbm, r_hbm, gsum_hbm,
          sidx0, sidx1, ridx0, ridx1, srow0, srow1, rrow0, rrow1,
          gsem, wsem):
        wid = lax.axis_index("s") * _NC + lax.axis_index("c")
        base = wid * per_w
        sidx = (sidx0, sidx1)
        ridx = (ridx0, ridx1)
        srow = (srow0, srow1)
        rrow = (rrow0, rrow1)

        def issue(chunk, b):
            off = base + chunk * blk
            pltpu.sync_copy(s_hbm.at[pl.ds(e0 + off, blk)], sidx[b])
            pltpu.sync_copy(r_hbm.at[pl.ds(e0 + off, blk)], ridx[b])
            pltpu.async_copy(xs_hbm.at[sidx[b]], srow[b], gsem)
            pltpu.async_copy(xr_hbm.at[ridx[b]], rrow[b], gsem)

        def wait_gather(b):
            pltpu.make_async_copy(xs_hbm.at[sidx[b]], srow[b], gsem).wait()
            pltpu.make_async_copy(xr_hbm.at[ridx[b]], rrow[b], gsem).wait()

        def accumulate(b):
            sb, rb = srow[b], rrow[b]

            @pl.loop(0, blk)
            def _(r):
                for j in range(d // 16):
                    sl = pl.ds(j * 16, 16)
                    sb[r, sl] = sb[r, sl] + rb[r, sl]

        def writeback(chunk, b):
            off = base + chunk * blk
            pltpu.async_copy(srow[b], gsum_hbm.at[pl.ds(off, blk)], wsem)

        def wait_writeback(chunk, b):
            off = base + chunk * blk
            pltpu.make_async_copy(
                srow[b], gsum_hbm.at[pl.ds(off, blk)], wsem).wait()

        issue(0, 0)
        issue(1, 1)

        @pl.loop(0, chunks - 2, step=2)
        def _(ci):
            wait_gather(0)
            accumulate(0)
            writeback(ci, 0)
            wait_gather(1)
            accumulate(1)
            writeback(ci + 1, 1)
            wait_writeback(ci, 0)
            issue(ci + 2, 0)
            wait_writeback(ci + 1, 1)

            @pl.when(ci + 3 < chunks)
            def _():
                issue(ci + 3, 1)

        wait_gather(0)
        accumulate(0)
        writeback(chunks - 1, 0)
        wait_writeback(chunks - 1, 0)

    return k(xs, xr, senders, receivers)


def _edge_body(g_ref, ef_ref, w1_ref, b1_ref, w2_ref, b2_ref, out_ref):
    ef = ef_ref[...]
    h = g_ref[...] + b1_ref[...] + jnp.dot(
        ef.astype(jnp.bfloat16), w1_ref[...],
        preferred_element_type=jnp.float32)
    h = jnp.maximum(h, 0.0)
    out_ref[...] = ef + b2_ref[...] + jnp.dot(
        h.astype(jnp.bfloat16), w2_ref[...],
        preferred_element_type=jnp.float32)


def _edge(gsum, ef, w1, b1, w2, b2, e0, e_seg):
    d = ef.shape[1]
    blk = 2000
    grid = e_seg // blk
    assert grid * blk == e_seg and e0 % blk == 0
    seg = e0 // blk
    row = pl.BlockSpec((blk, d), lambda i: (i, 0))
    row_seg = pl.BlockSpec((blk, d), lambda i: (i + seg, 0))
    full = pl.BlockSpec((d, d), lambda i: (0, 0))
    vec = pl.BlockSpec((1, d), lambda i: (0, 0))
    return pl.pallas_call(
        _edge_body,
        grid=(grid,),
        in_specs=[row, row_seg, full, vec, full, vec],
        out_specs=row,
        out_shape=jax.ShapeDtypeStruct((e_seg, d), jnp.float32),
    )(gsum, ef, w1, b1, w2, b2)


def _scatter(ne_a, ne_b, receivers, zeros_nd):
    per_core, d = ne_a.shape
    n = zeros_nd.shape[0]
    blk = 80
    per_tile = per_core // _NS
    chunks = per_tile // blk
    rows_per_tile = n // _NS
    assert chunks * blk == per_tile and rows_per_tile * _NS == n
    assert chunks % 2 == 1
    assert rows_per_tile % 8 == 0

    mesh = plsc.VectorSubcoreMesh(core_axis_name="c", subcore_axis_name="s")
    out = jax.ShapeDtypeStruct((n, d), jnp.float32)
    out_ne = jax.ShapeDtypeStruct((2 * per_core, d), jnp.float32)
    idx_t = pltpu.VMEM((blk,), jnp.int32)
    row_t = pltpu.VMEM((blk, d), jnp.float32)

    @functools.partial(
        pl.kernel,
        out_type=(out, out, out_ne),
        mesh=mesh,
        scratch_types=[
            pltpu.VMEM_SHARED((n, d), jnp.float32),
            idx_t, idx_t, row_t, row_t,
            pltpu.SemaphoreType.DMA, pltpu.SemaphoreType.DMA,
            pltpu.SemaphoreType.DMA,
        ],
    )
    def k(nea_hbm, neb_hbm, r_hbm, z_hbm, a0_hbm, a1_hbm, ne_hbm, shared,
          eidx0, eidx1, erow0, erow1, lsem, asem, wsem):
        c = lax.axis_index("c")
        s = lax.axis_index("s")
        r0 = s * rows_per_tile
        pltpu.sync_copy(z_hbm.at[pl.ds(r0, rows_per_tile)],
                        shared.at[pl.ds(r0, rows_per_tile)])
        plsc.subcore_barrier()

        base = s * per_tile
        eidx = (eidx0, eidx1)
        erow = (erow0, erow1)

        def load(chunk, b):
            off = base + chunk * blk
            pltpu.async_copy(r_hbm.at[pl.ds(c * per_core + off, blk)],
                             eidx[b], lsem)

            @pl.when(c == 0)
            def _():
                pltpu.async_copy(nea_hbm.at[pl.ds(off, blk)], erow[b], lsem)

            @pl.when(c == 1)
            def _():
                pltpu.async_copy(neb_hbm.at[pl.ds(off, blk)], erow[b], lsem)

        def wait_load(chunk, b):
            off = base + chunk * blk
            pltpu.make_async_copy(
                r_hbm.at[pl.ds(off, blk)], eidx[b], lsem).wait()
            pltpu.make_async_copy(
                nea_hbm.at[pl.ds(off, blk)], erow[b], lsem).wait()

        def add(b):
            pltpu.async_copy(erow[b], shared.at[eidx[b]], asem, add=True)

        def wait_add(b):
            pltpu.make_async_copy(erow[b], shared.at[eidx[b]], asem).wait()

        def emit(chunk, b):
            off = c * per_core + base + chunk * blk
            pltpu.async_copy(erow[b], ne_hbm.at[pl.ds(off, blk)], wsem)

        def wait_emit(chunk, b):
            off = c * per_core + base + chunk * blk
            pltpu.make_async_copy(
                erow[b], ne_hbm.at[pl.ds(off, blk)], wsem).wait()

        load(0, 0)
        load(1, 1)

        @pl.loop(0, chunks - 2, step=2)
        def _(ci):
            wait_load(ci, 0)
            add(0)
            emit(ci, 0)
            wait_load(ci + 1, 1)
            add(1)
            emit(ci + 1, 1)
            wait_add(0)
            wait_emit(ci, 0)
            load(ci + 2, 0)
            wait_add(1)

            @pl.when(ci + 3 < chunks)
            def _():
                wait_emit(ci + 1, 1)
                load(ci + 3, 1)

        wait_emit(chunks - 2, 1)
        wait_load(chunks - 1, 0)
        add(0)
        emit(chunks - 1, 0)
        wait_add(0)
        wait_emit(chunks - 1, 0)

        plsc.subcore_barrier()

        @pl.when(c == 0)
        def _():
            pltpu.sync_copy(shared.at[pl.ds(r0, rows_per_tile)],
                            a0_hbm.at[pl.ds(r0, rows_per_tile)])

        @pl.when(c == 1)
        def _():
            pltpu.sync_copy(shared.at[pl.ds(r0, rows_per_tile)],
                            a1_hbm.at[pl.ds(r0, rows_per_tile)])

    return k(ne_a, ne_b, receivers, zeros_nd)


def _node_body(x_ref, a0_ref, a1_ref, w1x_ref, w1a_ref, b1_ref, w2_ref,
               b2_ref, out_ref):
    x = x_ref[...]
    agg = a0_ref[...] + a1_ref[...]
    h = b1_ref[...] + jnp.dot(x, w1x_ref[...],
                              preferred_element_type=jnp.float32)
    h = h + jnp.dot(agg, w1a_ref[...], preferred_element_type=jnp.float32)
    h = jnp.maximum(h, 0.0)
    out_ref[...] = x + b2_ref[...] + jnp.dot(
        h, w2_ref[...], preferred_element_type=jnp.float32)


def _node(x, a0, a1, w1x, w1a, b1, w2, b2):
    n, d = x.shape
    blk = 1000
    grid = n // blk
    assert grid * blk == n
    row = pl.BlockSpec((blk, d), lambda i: (i, 0))
    full = pl.BlockSpec((d, d), lambda i: (0, 0))
    vec = pl.BlockSpec((1, d), lambda i: (0, 0))
    return pl.pallas_call(
        _node_body,
        grid=(grid,),
        in_specs=[row, row, row, full, full, vec, full, vec],
        out_specs=row,
        out_shape=jax.ShapeDtypeStruct((n, d), jnp.float32),
    )(x, a0, a1, w1x, w1a, b1, w2, b2)


def kernel(x, edge_features, W1_e, b1_e, W2_e, b2_e,
           W1_n, b1_n, W2_n, b2_n, senders, receivers):
    n, d = x.shape
    e = senders.shape[0]
    senders = senders.astype(jnp.int32)
    receivers = receivers.astype(jnp.int32)
    w1_s, w1_r, w1_ef = W1_e[:d], W1_e[d:2 * d], W1_e[2 * d:]
    w1_ef_b = w1_ef.astype(jnp.bfloat16)
    w2_e_b = W2_e.astype(jnp.bfloat16)
    b1_e2 = b1_e.reshape(1, d)
    b2_e2 = b2_e.reshape(1, d)
    b1_n2 = b1_n.reshape(1, d)
    b2_n2 = b2_n.reshape(1, d)
    w1_nx, w1_na = W1_n[:d], W1_n[d:]
    n_pad = ((n + 8 * _NS - 1) // (8 * _NS)) * (8 * _NS)
    zeros_nd = jnp.zeros((n_pad, d), jnp.float32)
    half = e // 2

    xs, xr = _pre(x, w1_s, w1_r)
    g_a = _gather_sum(xs, xr, senders, receivers, 0, half)
    ne_a = _edge(g_a, edge_features, w1_ef_b, b1_e2, w2_e_b, b2_e2, 0, half)
    g_b = _gather_sum(xs, xr, senders, receivers, half, half)
    ne_b = _edge(g_b, edge_features, w1_ef_b, b1_e2, w2_e_b, b2_e2,
                 half, half)
    a0, a1, new_e = _scatter(ne_a, ne_b, receivers, zeros_nd)
    new_x = _node(x, a0, a1, w1_nx, w1_na, b1_n2, W2_n, b2_n2)
    return (new_x, new_e)


# speedup vs baseline: 1.2547x; 1.0554x over previous
import functools

import jax
import jax.numpy as jnp
from jax import lax
from jax.experimental import pallas as pl
from jax.experimental.pallas import tpu as pltpu
from jax.experimental.pallas import tpu_sc as plsc

_NC = 2
_NS = 16
_NW = _NC * _NS


def _pre_body(x_ref, ws_ref, wr_ref, xs_ref, xr_ref):
    x = x_ref[...]
    xs_ref[...] = jnp.dot(x, ws_ref[...], preferred_element_type=jnp.float32)
    xr_ref[...] = jnp.dot(x, wr_ref[...], preferred_element_type=jnp.float32)


def _pre(x, w_s, w_r):
    n, d = x.shape
    out = jax.ShapeDtypeStruct((n, d), jnp.float32)
    return pl.pallas_call(
        _pre_body,
        out_shape=(out, out),
    )(x, w_s, w_r)


def _gather_sum(xs, xr, senders, receivers, e0, e_seg):
    d = xs.shape[1]
    blk = 200
    per_w = e_seg // _NW
    chunks = per_w // blk
    assert chunks * blk == per_w and per_w * _NW == e_seg
    assert chunks % 2 == 1

    mesh = plsc.VectorSubcoreMesh(core_axis_name="c", subcore_axis_name="s")
    out = jax.ShapeDtypeStruct((e_seg, d), jnp.float32)
    idx_t = pltpu.VMEM((blk,), jnp.int32)
    row_t = pltpu.VMEM((blk, d), jnp.float32)

    @functools.partial(
        pl.kernel,
        out_type=out,
        mesh=mesh,
        scratch_types=[idx_t, idx_t, idx_t, idx_t, row_t, row_t, row_t, row_t,
                       pltpu.SemaphoreType.DMA, pltpu.SemaphoreType.DMA],
    )
    def k(xs_hbm, xr_hbm, s_hbm, r_hbm, gsum_hbm,
          sidx0, sidx1, ridx0, ridx1, srow0, srow1, rrow0, rrow1,
          gsem, wsem):
        wid = lax.axis_index("s") * _NC + lax.axis_index("c")
        base = wid * per_w
        sidx = (sidx0, sidx1)
        ridx = (ridx0, ridx1)
        srow = (srow0, srow1)
        rrow = (rrow0, rrow1)

        def issue(chunk, b):
            off = base + chunk * blk
            pltpu.sync_copy(s_hbm.at[pl.ds(e0 + off, blk)], sidx[b])
            pltpu.sync_copy(r_hbm.at[pl.ds(e0 + off, blk)], ridx[b])
            pltpu.async_copy(xs_hbm.at[sidx[b]], srow[b], gsem)
            pltpu.async_copy(xr_hbm.at[ridx[b]], rrow[b], gsem)

        def wait_gather(b):
            pltpu.make_async_copy(xs_hbm.at[sidx[b]], srow[b], gsem).wait()
            pltpu.make_async_copy(xr_hbm.at[ridx[b]], rrow[b], gsem).wait()

        def accumulate(b):
            sb, rb = srow[b], rrow[b]

            @pl.loop(0, blk)
            def _(r):
                for j in range(d // 16):
                    sl = pl.ds(j * 16, 16)
                    sb[r, sl] = sb[r, sl] + rb[r, sl]

        def writeback(chunk, b):
            off = base + chunk * blk
            pltpu.async_copy(srow[b], gsum_hbm.at[pl.ds(off, blk)], wsem)

        def wait_writeback(chunk, b):
            off = base + chunk * blk
            pltpu.make_async_copy(
                srow[b], gsum_hbm.at[pl.ds(off, blk)], wsem).wait()

        issue(0, 0)
        issue(1, 1)

        @pl.loop(0, chunks - 2, step=2)
        def _(ci):
            wait_gather(0)
            accumulate(0)
            writeback(ci, 0)
            wait_gather(1)
            accumulate(1)
            writeback(ci + 1, 1)
            wait_writeback(ci, 0)
            issue(ci + 2, 0)
            wait_writeback(ci + 1, 1)

            @pl.when(ci + 3 < chunks)
            def _():
                issue(ci + 3, 1)

        wait_gather(0)
        accumulate(0)
        writeback(chunks - 1, 0)
        wait_writeback(chunks - 1, 0)

    return k(xs, xr, senders, receivers)


def _edge_body(g_ref, ef_ref, w1_ref, b1_ref, w2_ref, b2_ref, out_ref):
    ef = ef_ref[...]
    h = g_ref[...] + b1_ref[...] + jnp.dot(
        ef.astype(jnp.bfloat16), w1_ref[...],
        preferred_element_type=jnp.float32)
    h = jnp.maximum(h, 0.0)
    out_ref[...] = ef + b2_ref[...] + jnp.dot(
        h.astype(jnp.bfloat16), w2_ref[...],
        preferred_element_type=jnp.float32)


def _edge_alias_body(g_ref, ef_ref, w1_ref, b1_ref, w2_ref, b2_ref, _prev,
                     out_ref):
    _edge_body(g_ref, ef_ref, w1_ref, b1_ref, w2_ref, b2_ref, out_ref)


def _edge(gsum, ef, w1, b1, w2, b2, e0, e_seg, prev=None):
    e_total, d = ef.shape
    blk = 2000
    grid = e_seg // blk
    assert grid * blk == e_seg and e0 % blk == 0
    seg = e0 // blk
    row_loc = pl.BlockSpec((blk, d), lambda i: (i, 0))
    row_seg = pl.BlockSpec((blk, d), lambda i: (i + seg, 0))
    full = pl.BlockSpec((d, d), lambda i: (0, 0))
    vec = pl.BlockSpec((1, d), lambda i: (0, 0))
    out_shape = jax.ShapeDtypeStruct((e_total, d), jnp.float32)
    if prev is None:
        return pl.pallas_call(
            _edge_body,
            grid=(grid,),
            in_specs=[row_loc, row_seg, full, vec, full, vec],
            out_specs=row_seg,
            out_shape=out_shape,
        )(gsum, ef, w1, b1, w2, b2)
    hbm = pl.BlockSpec(memory_space=pltpu.MemorySpace.HBM)
    return pl.pallas_call(
        _edge_alias_body,
        grid=(grid,),
        in_specs=[row_loc, row_seg, full, vec, full, vec, hbm],
        out_specs=row_seg,
        out_shape=out_shape,
        input_output_aliases={6: 0},
    )(gsum, ef, w1, b1, w2, b2, prev)


def _scatter(ne, receivers, zeros_nd):
    e, d = ne.shape
    per_core = e // _NC
    n = zeros_nd.shape[0]
    blk = 80
    per_tile = per_core // _NS
    chunks = per_tile // blk
    rows_per_tile = n // _NS
    assert chunks * blk == per_tile and rows_per_tile * _NS == n
    assert chunks % 2 == 1
    assert rows_per_tile % 8 == 0

    mesh = plsc.VectorSubcoreMesh(core_axis_name="c", subcore_axis_name="s")
    out = jax.ShapeDtypeStruct((n, d), jnp.float32)
    idx_t = pltpu.VMEM((blk,), jnp.int32)
    row_t = pltpu.VMEM((blk, d), jnp.float32)

    @functools.partial(
        pl.kernel,
        out_type=(out, out),
        mesh=mesh,
        scratch_types=[
            pltpu.VMEM_SHARED((n, d), jnp.float32),
            idx_t, idx_t, row_t, row_t,
            pltpu.SemaphoreType.DMA, pltpu.SemaphoreType.DMA,
        ],
    )
    def k(ne_hbm, r_hbm, z_hbm, a0_hbm, a1_hbm, shared,
          eidx0, eidx1, erow0, erow1, lsem, asem):
        c = lax.axis_index("c")
        s = lax.axis_index("s")
        r0 = s * rows_per_tile
        pltpu.sync_copy(z_hbm.at[pl.ds(r0, rows_per_tile)],
                        shared.at[pl.ds(r0, rows_per_tile)])
        plsc.subcore_barrier()

        base = c * per_core + s * per_tile
        eidx = (eidx0, eidx1)
        erow = (erow0, erow1)

        def load(chunk, b):
            off = base + chunk * blk
            pltpu.async_copy(r_hbm.at[pl.ds(off, blk)], eidx[b], lsem)
            pltpu.async_copy(ne_hbm.at[pl.ds(off, blk)], erow[b], lsem)

        def wait_load(chunk, b):
            off = base + chunk * blk
            pltpu.make_async_copy(
                r_hbm.at[pl.ds(off, blk)], eidx[b], lsem).wait()
            pltpu.make_async_copy(
                ne_hbm.at[pl.ds(off, blk)], erow[b], lsem).wait()

        def add(b):
            pltpu.async_copy(erow[b], shared.at[eidx[b]], asem, add=True)

        def wait_add(b):
            pltpu.make_async_copy(erow[b], shared.at[eidx[b]], asem).wait()

        load(0, 0)
        load(1, 1)

        @pl.loop(0, chunks - 2, step=2)
        def _(ci):
            wait_load(ci, 0)
            add(0)
            wait_load(ci + 1, 1)
            add(1)
            wait_add(0)
            load(ci + 2, 0)
            wait_add(1)

            @pl.when(ci + 3 < chunks)
            def _():
                load(ci + 3, 1)

        wait_load(chunks - 1, 0)
        add(0)
        wait_add(0)

        plsc.subcore_barrier()

        @pl.when(c == 0)
        def _():
            pltpu.sync_copy(shared.at[pl.ds(r0, rows_per_tile)],
                            a0_hbm.at[pl.ds(r0, rows_per_tile)])

        @pl.when(c == 1)
        def _():
            pltpu.sync_copy(shared.at[pl.ds(r0, rows_per_tile)],
                            a1_hbm.at[pl.ds(r0, rows_per_tile)])

    return k(ne, receivers, zeros_nd)


def _node_body(x_ref, a0_ref, a1_ref, w1x_ref, w1a_ref, b1_ref, w2_ref,
               b2_ref, out_ref):
    x = x_ref[...]
    agg = a0_ref[...] + a1_ref[...]
    h = b1_ref[...] + jnp.dot(x, w1x_ref[...],
                              preferred_element_type=jnp.float32)
    h = h + jnp.dot(agg, w1a_ref[...], preferred_element_type=jnp.float32)
    h = jnp.maximum(h, 0.0)
    out_ref[...] = x + b2_ref[...] + jnp.dot(
        h, w2_ref[...], preferred_element_type=jnp.float32)


def _node(x, a0, a1, w1x, w1a, b1, w2, b2):
    n, d = x.shape
    blk = 1000
    grid = n // blk
    assert grid * blk == n
    row = pl.BlockSpec((blk, d), lambda i: (i, 0))
    full = pl.BlockSpec((d, d), lambda i: (0, 0))
    vec = pl.BlockSpec((1, d), lambda i: (0, 0))
    return pl.pallas_call(
        _node_body,
        grid=(grid,),
        in_specs=[row, row, row, full, full, vec, full, vec],
        out_specs=row,
        out_shape=jax.ShapeDtypeStruct((n, d), jnp.float32),
    )(x, a0, a1, w1x, w1a, b1, w2, b2)


def kernel(x, edge_features, W1_e, b1_e, W2_e, b2_e,
           W1_n, b1_n, W2_n, b2_n, senders, receivers):
    n, d = x.shape
    e = senders.shape[0]
    senders = senders.astype(jnp.int32)
    receivers = receivers.astype(jnp.int32)
    w1_s, w1_r, w1_ef = W1_e[:d], W1_e[d:2 * d], W1_e[2 * d:]
    w1_ef_b = w1_ef.astype(jnp.bfloat16)
    w2_e_b = W2_e.astype(jnp.bfloat16)
    b1_e2 = b1_e.reshape(1, d)
    b2_e2 = b2_e.reshape(1, d)
    b1_n2 = b1_n.reshape(1, d)
    b2_n2 = b2_n.reshape(1, d)
    w1_nx, w1_na = W1_n[:d], W1_n[d:]
    n_pad = ((n + 8 * _NS - 1) // (8 * _NS)) * (8 * _NS)
    zeros_nd = jnp.zeros((n_pad, d), jnp.float32)
    half = e // 2

    xs, xr = _pre(x, w1_s, w1_r)
    g_a = _gather_sum(xs, xr, senders, receivers, 0, half)
    ne_a = _edge(g_a, edge_features, w1_ef_b, b1_e2, w2_e_b, b2_e2, 0, half)
    g_b = _gather_sum(xs, xr, senders, receivers, half, half)
    new_e = _edge(g_b, edge_features, w1_ef_b, b1_e2, w2_e_b, b2_e2,
                  half, half, prev=ne_a)
    a0, a1 = _scatter(new_e, receivers, zeros_nd)
    new_x = _node(x, a0, a1, w1_nx, w1_na, b1_n2, W2_n, b2_n2)
    return (new_x, new_e)
